# baseline (speedup 1.0000x reference)
import jax
import jax.numpy as jnp
from jax import lax
from jax.experimental import pallas as pl
from jax.experimental.pallas import tpu as pltpu
from jax.experimental.pallas import tpu_sc as plsc

_NC, _NS = 2, 16
_NW = _NC * _NS
_CH = 4
_B, _S, _D = 4, 4096, 2048
_ROWS = _S // _NW
_NCHUNK = _ROWS // _CH


def _sc_body(x_hbm, pos_hbm, out_hbm,
             xin0, xin1, xout0, xout1, posb0, posb1,
             sin0, sin1, sout0, sout1, spos0, spos1):
    xin = (xin0, xin1)
    xout = (xout0, xout1)
    posb = (posb0, posb1)
    sin = (sin0, sin1)
    sout = (sout0, sout1)
    spos = (spos0, spos1)

    wid = lax.axis_index("s") * _NC + lax.axis_index("c")
    base = wid * _ROWS

    def row_off(c, b):
        return b * _S + base + c * _CH

    def start_load(c, b, p):
        pltpu.async_copy(x_hbm.at[pl.ds(row_off(c, b), _CH), :], xin[p], sin[p])

    def wait_load(p):
        pltpu.make_async_copy(x_hbm.at[pl.ds(0, _CH), :], xin[p], sin[p]).wait()

    def start_store(c, b, p):
        pltpu.async_copy(xout[p], out_hbm.at[pl.ds(row_off(c, b), _CH), :], sout[p])

    def wait_store(p):
        pltpu.make_async_copy(x_hbm.at[pl.ds(0, _CH), :], xout[p], sout[p]).wait()

    def start_pos(c, pp):
        pltpu.async_copy(pos_hbm.at[pl.ds(base + c * _CH, _CH), :], posb[pp], spos[pp])

    def wait_pos(pp):
        pltpu.make_async_copy(pos_hbm.at[pl.ds(0, _CH), :], posb[pp], spos[pp]).wait()

    start_load(0, 0, 0)
    start_load(0, 1, 1)
    start_pos(0, 0)
    start_pos(1, 1)

    def q_body(q, carry):
        for cc in range(2):
            c = 2 * q + cc
            wait_pos(cc)
            for b in range(4):
                p = b % 2
                wait_load(p)

                if b < 2:
                    @pl.when(c > 0)
                    def _():
                        wait_store(p)
                else:
                    wait_store(p)

                src, dst, pv = xin[p], xout[p], posb[cc]

                @plsc.parallel_loop(0, _D, step=16, unroll=2)
                def add_j(j):
                    sl = pl.ds(j, 16)
                    for r in range(_CH):
                        dst[r, sl] = src[r, sl] + pv[r, sl]

                if b < 2:
                    start_load(c, b + 2, p)
                else:
                    @pl.when(c < _NCHUNK - 1)
                    def _():
                        start_load(c + 1, b - 2, p)

                start_store(c, b, p)

            @pl.when(c + 2 < _NCHUNK)
            def _():
                start_pos(c + 2, cc)
        return carry

    lax.fori_loop(0, _NCHUNK // 2, q_body, 0)

    wait_store(0)
    wait_store(1)


def kernel(x, pos_table):
    B, S, D = x.shape
    run = pl.kernel(
        _sc_body,
        out_type=jax.ShapeDtypeStruct((B * S, D), x.dtype),
        mesh=plsc.VectorSubcoreMesh(core_axis_name="c", subcore_axis_name="s"),
        scratch_types=(
            [pltpu.VMEM((_CH, _D), jnp.float32) for _ in range(6)]
            + [pltpu.SemaphoreType.DMA for _ in range(6)]
        ),
    )
    out = run(x.reshape(B * S, D), pos_table)
    return out.reshape(B, S, D)


# speedup vs baseline: 1.1067x; 1.1067x over previous
import jax
import jax.numpy as jnp
from jax import lax
from jax.experimental import pallas as pl
from jax.experimental.pallas import tpu as pltpu
from jax.experimental.pallas import tpu_sc as plsc

_NC, _NS = 2, 16
_NW = _NC * _NS
_CH = 8
_B, _S, _D = 4, 4096, 2048
_ROWS = _S // _NW
_NCHUNK = _ROWS // _CH


def _sc_body(x_hbm, pos_hbm, out_hbm,
             xin0, xin1, xout0, xout1, posb0, posb1,
             sin0, sin1, sout0, sout1, spos0, spos1):
    xin = (xin0, xin1)
    xout = (xout0, xout1)
    posb = (posb0, posb1)
    sin = (sin0, sin1)
    sout = (sout0, sout1)
    spos = (spos0, spos1)

    wid = lax.axis_index("s") * _NC + lax.axis_index("c")
    base = wid * _ROWS

    def row_off(c, b):
        return b * _S + base + c * _CH

    def start_load(c, b, p):
        pltpu.async_copy(x_hbm.at[pl.ds(row_off(c, b), _CH), :], xin[p], sin[p])

    def wait_load(p):
        pltpu.make_async_copy(x_hbm.at[pl.ds(0, _CH), :], xin[p], sin[p]).wait()

    def start_store(c, b, p):
        pltpu.async_copy(xout[p], out_hbm.at[pl.ds(row_off(c, b), _CH), :], sout[p])

    def wait_store(p):
        pltpu.make_async_copy(x_hbm.at[pl.ds(0, _CH), :], xout[p], sout[p]).wait()

    def start_pos(c, pp):
        pltpu.async_copy(pos_hbm.at[pl.ds(base + c * _CH, _CH), :], posb[pp], spos[pp])

    def wait_pos(pp):
        pltpu.make_async_copy(pos_hbm.at[pl.ds(0, _CH), :], posb[pp], spos[pp]).wait()

    start_load(0, 0, 0)
    start_load(0, 1, 1)
    start_pos(0, 0)
    start_pos(1, 1)

    def q_body(q, carry):
        for cc in range(2):
            c = 2 * q + cc
            wait_pos(cc)
            for b in range(4):
                p = b % 2
                wait_load(p)

                if b < 2:
                    @pl.when(c > 0)
                    def _():
                        wait_store(p)
                else:
                    wait_store(p)

                src, dst, pv = xin[p], xout[p], posb[cc]

                @plsc.parallel_loop(0, _D, step=16, unroll=4)
                def add_j(j):
                    sl = pl.ds(j, 16)
                    for r in range(_CH):
                        dst[r, sl] = src[r, sl] + pv[r, sl]

                if b < 2:
                    start_load(c, b + 2, p)
                else:
                    @pl.when(c < _NCHUNK - 1)
                    def _():
                        start_load(c + 1, b - 2, p)

                start_store(c, b, p)

            @pl.when(c + 2 < _NCHUNK)
            def _():
                start_pos(c + 2, cc)
        return carry

    lax.fori_loop(0, _NCHUNK // 2, q_body, 0)

    wait_store(0)
    wait_store(1)


def kernel(x, pos_table):
    B, S, D = x.shape
    run = pl.kernel(
        _sc_body,
        out_type=jax.ShapeDtypeStruct((B * S, D), x.dtype),
        mesh=plsc.VectorSubcoreMesh(core_axis_name="c", subcore_axis_name="s"),
        scratch_types=(
            [pltpu.VMEM((_CH, _D), jnp.float32) for _ in range(6)]
            + [pltpu.SemaphoreType.DMA for _ in range(6)]
        ),
    )
    out = run(x.reshape(B * S, D), pos_table)
    return out.reshape(B, S, D)


# speedup vs baseline: 1.1213x; 1.0133x over previous
import jax
import jax.numpy as jnp
from jax import lax
from jax.experimental import pallas as pl
from jax.experimental.pallas import tpu as pltpu
from jax.experimental.pallas import tpu_sc as plsc

_NC, _NS = 2, 16
_NW = _NC * _NS
_CH = 8
_B, _S, _D = 4, 4096, 2048
_ROWS = _S // _NW
_NCHUNK = _ROWS // _CH


def _sc_body(x_hbm, pos_hbm, out_hbm,
             xin0, xin1, xout0, xout1, posb0, posb1,
             sin0, sin1, sout0, sout1, spos0, spos1):
    xin = (xin0, xin1)
    xout = (xout0, xout1)
    posb = (posb0, posb1)
    sin = (sin0, sin1)
    sout = (sout0, sout1)
    spos = (spos0, spos1)

    wid = lax.axis_index("s") * _NC + lax.axis_index("c")
    base = wid * _ROWS

    def row_off(c, b):
        return b * _S + base + c * _CH

    def start_load(c, b, p):
        pltpu.async_copy(x_hbm.at[pl.ds(row_off(c, b), _CH), :], xin[p], sin[p])

    def wait_load(p):
        pltpu.make_async_copy(x_hbm.at[pl.ds(0, _CH), :], xin[p], sin[p]).wait()

    def start_store(c, b, p):
        pltpu.async_copy(xout[p], out_hbm.at[pl.ds(row_off(c, b), _CH), :], sout[p])

    def wait_store(p):
        pltpu.make_async_copy(x_hbm.at[pl.ds(0, _CH), :], xout[p], sout[p]).wait()

    def start_pos(c, pp):
        pltpu.async_copy(pos_hbm.at[pl.ds(base + c * _CH, _CH), :], posb[pp], spos[pp])

    def wait_pos(pp):
        pltpu.make_async_copy(pos_hbm.at[pl.ds(0, _CH), :], posb[pp], spos[pp]).wait()

    start_load(0, 0, 0)
    start_load(0, 1, 1)
    start_pos(0, 0)
    start_pos(1, 1)

    def q_body(q, carry):
        for cc in range(2):
            c = 2 * q + cc
            wait_pos(cc)
            for b in range(4):
                p = b % 2
                wait_load(p)

                if b < 2:
                    @pl.when(c > 0)
                    def _():
                        wait_store(p)
                else:
                    wait_store(p)

                src, dst, pv = xin[p], xout[p], posb[cc]

                @plsc.parallel_loop(0, _D, step=16, unroll=2)
                def add_j(j):
                    sl = pl.ds(j, 16)
                    for r in range(_CH):
                        dst[r, sl] = src[r, sl] + pv[r, sl]

                if b < 2:
                    start_load(c, b + 2, p)
                else:
                    @pl.when(c < _NCHUNK - 1)
                    def _():
                        start_load(c + 1, b - 2, p)

                start_store(c, b, p)

            @pl.when(c + 2 < _NCHUNK)
            def _():
                start_pos(c + 2, cc)
        return carry

    lax.fori_loop(0, _NCHUNK // 2, q_body, 0)

    wait_store(0)
    wait_store(1)


def kernel(x, pos_table):
    B, S, D = x.shape
    run = pl.kernel(
        _sc_body,
        out_type=jax.ShapeDtypeStruct((B * S, D), x.dtype),
        mesh=plsc.VectorSubcoreMesh(core_axis_name="c", subcore_axis_name="s"),
        scratch_types=(
            [pltpu.VMEM((_CH, _D), jnp.float32) for _ in range(6)]
            + [pltpu.SemaphoreType.DMA for _ in range(6)]
        ),
    )
    out = run(x.reshape(B * S, D), pos_table)
    return out.reshape(B, S, D)
